# baseline (speedup 1.0000x reference)
import functools

import jax
import jax.numpy as jnp
from jax import lax
from jax.experimental import pallas as pl
from jax.experimental.pallas import tpu as pltpu
from jax.experimental.pallas import tpu_sc as plsc

G0, G1, D = 128, 64, 256
L = 16
NC, NS = 2, 16
NW = NC * NS
SC_G0 = 64
QPW = SC_G0 // NW
KD = D // L
GPW = QPW * G1 // L

TC_G0 = G0 - SC_G0
TCB0 = 16

LEAF = (0, 8, 4, 12, 2, 10, 6, 14, 1, 9, 5, 13, 3, 11, 7, 15)

_mesh = plsc.VectorSubcoreMesh(core_axis_name="c", subcore_axis_name="s")


@functools.partial(
    pl.kernel,
    mesh=_mesh,
    out_type=jax.ShapeDtypeStruct((SC_G0, G1), jnp.float32),
    scratch_types=[
        pltpu.VMEM((D,), jnp.float32),
        pltpu.VMEM((QPW, G1, D), jnp.float32),
        pltpu.VMEM((QPW, G1), jnp.float32),
        pltpu.SemaphoreType.DMA,
        pltpu.SemaphoreType.DMA,
        pltpu.SemaphoreType.DMA,
        pltpu.SemaphoreType.DMA,
    ],
)
def _som_distances_sc(x_hbm, w_hbm, out_hbm, x_v, w_v, o_v, s0, s1, s2, s3):
    wid = lax.axis_index("s") * NC + lax.axis_index("c")
    q0 = wid * QPW
    sems = (s0, s1, s2, s3)
    H = G1 // 2
    cps = [
        pltpu.async_copy(
            w_hbm.at[q0 + c // 2, pl.ds((c & 1) * H, H)],
            w_v.at[c // 2, pl.ds((c & 1) * H, H)],
            sems[c],
        )
        for c in range(2 * QPW)
    ]
    pltpu.sync_copy(x_hbm, x_v)

    xs = [x_v[pl.ds(k * L, L)] for k in range(KD)]
    lanes = lax.iota(jnp.int32, L)
    masks = {s: (lanes & s) == 0 for s in (8, 4, 2, 1)}
    perms = {s: lanes ^ s for s in (8, 4, 2, 1)}

    def xperm(v, s):
        return v.at[perms[s]].get(mode="promise_in_bounds", unique_indices=True)

    def combine(a, b, s):
        return jnp.where(masks[s], a, xperm(b, s)) + jnp.where(
            masks[s], xperm(a, s), b
        )

    def acc_row(q, u):
        d = w_v[q, u, pl.ds(0, L)] - xs[0]
        acc = d * d
        for k in range(1, KD):
            d = w_v[q, u, pl.ds(k * L, L)] - xs[k]
            acc = acc + d * d
        return acc

    def group_body(g, carry):
        q = g >> 2
        c0 = (g & 3) << 4

        def quad(i):
            t8a = combine(
                acc_row(q, c0 + LEAF[4 * i]), acc_row(q, c0 + LEAF[4 * i + 1]), 8
            )
            t8b = combine(
                acc_row(q, c0 + LEAF[4 * i + 2]),
                acc_row(q, c0 + LEAF[4 * i + 3]),
                8,
            )
            return combine(t8a, t8b, 4)

        for b in (1, 2, 3):

            @pl.when(g == 2 * b)
            def _(b=b):
                cps[b].wait()

        t2a = combine(quad(0), quad(1), 2)
        t2b = combine(quad(2), quad(3), 2)
        o_v[q, pl.ds(c0, L)] = combine(t2a, t2b, 1)
        return carry

    cps[0].wait()
    lax.fori_loop(0, GPW, group_body, 0)
    pltpu.sync_copy(o_v, out_hbm.at[pl.ds(q0, QPW)])


def _som_tc_body(x_ref, w_ref, o_ref):
    d = w_ref[...] - x_ref[...]
    o_ref[...] = jnp.sum(d * d, axis=2)


_som_distances_tc = pl.pallas_call(
    _som_tc_body,
    grid=(TC_G0 // TCB0,),
    in_specs=[
        pl.BlockSpec((1, 1, D), lambda i: (0, 0, 0)),
        pl.BlockSpec((TCB0, G1, D), lambda i: (SC_G0 // TCB0 + i, 0, 0)),
    ],
    out_specs=pl.BlockSpec((TCB0, G1), lambda i: (i, 0)),
    out_shape=jax.ShapeDtypeStruct((TC_G0, G1), jnp.float32),
)


def kernel(x, weights):
    sc_out = _som_distances_sc(x, weights)
    tc_out = _som_distances_tc(x.reshape(1, 1, D), weights)
    return jnp.concatenate([sc_out, tc_out], axis=0)


# speedup vs baseline: 1.0480x; 1.0480x over previous
import functools

import jax
import jax.numpy as jnp
from jax import lax
from jax.experimental import pallas as pl
from jax.experimental.pallas import tpu as pltpu
from jax.experimental.pallas import tpu_sc as plsc

G0, G1, D = 128, 64, 256
L = 16
NC, NS = 2, 16
NW = NC * NS
SC_G0 = 32
QPW = SC_G0 // NW
KD = D // L
GPW = QPW * G1 // L

TC_G0 = G0 - SC_G0
TCB0 = 16

LEAF = (0, 8, 4, 12, 2, 10, 6, 14, 1, 9, 5, 13, 3, 11, 7, 15)

_mesh = plsc.VectorSubcoreMesh(core_axis_name="c", subcore_axis_name="s")


@functools.partial(
    pl.kernel,
    mesh=_mesh,
    out_type=jax.ShapeDtypeStruct((SC_G0, G1), jnp.float32),
    scratch_types=[
        pltpu.VMEM((D,), jnp.float32),
        pltpu.VMEM((QPW, G1, D), jnp.float32),
        pltpu.VMEM((QPW, G1), jnp.float32),
        pltpu.SemaphoreType.DMA,
        pltpu.SemaphoreType.DMA,
        pltpu.SemaphoreType.DMA,
        pltpu.SemaphoreType.DMA,
    ],
)
def _som_distances_sc(x_hbm, w_hbm, out_hbm, x_v, w_v, o_v, s0, s1, s2, s3):
    wid = lax.axis_index("s") * NC + lax.axis_index("c")
    q0 = wid * QPW
    sems = (s0, s1, s2, s3)
    H = G1 // 2
    cps = [
        pltpu.async_copy(
            w_hbm.at[q0 + c // 2, pl.ds((c & 1) * H, H)],
            w_v.at[c // 2, pl.ds((c & 1) * H, H)],
            sems[c],
        )
        for c in range(2 * QPW)
    ]
    pltpu.sync_copy(x_hbm, x_v)

    xs = [x_v[pl.ds(k * L, L)] for k in range(KD)]
    lanes = lax.iota(jnp.int32, L)
    masks = {s: (lanes & s) == 0 for s in (8, 4, 2, 1)}
    perms = {s: lanes ^ s for s in (8, 4, 2, 1)}

    def xperm(v, s):
        return v.at[perms[s]].get(mode="promise_in_bounds", unique_indices=True)

    def combine(a, b, s):
        return jnp.where(masks[s], a, xperm(b, s)) + jnp.where(
            masks[s], xperm(a, s), b
        )

    def acc_row(q, u):
        d = w_v[q, u, pl.ds(0, L)] - xs[0]
        acc = d * d
        for k in range(1, KD):
            d = w_v[q, u, pl.ds(k * L, L)] - xs[k]
            acc = acc + d * d
        return acc

    def group_body(g, carry):
        q = g >> 2
        c0 = (g & 3) << 4

        def quad(i):
            t8a = combine(
                acc_row(q, c0 + LEAF[4 * i]), acc_row(q, c0 + LEAF[4 * i + 1]), 8
            )
            t8b = combine(
                acc_row(q, c0 + LEAF[4 * i + 2]),
                acc_row(q, c0 + LEAF[4 * i + 3]),
                8,
            )
            return combine(t8a, t8b, 4)

        for b in range(1, 2 * QPW):

            @pl.when(g == 2 * b)
            def _(b=b):
                cps[b].wait()

        t2a = combine(quad(0), quad(1), 2)
        t2b = combine(quad(2), quad(3), 2)
        o_v[q, pl.ds(c0, L)] = combine(t2a, t2b, 1)
        return carry

    cps[0].wait()
    lax.fori_loop(0, GPW, group_body, 0)
    pltpu.sync_copy(o_v, out_hbm.at[pl.ds(q0, QPW)])


def _som_tc_body(x_ref, w_ref, o_ref):
    d = w_ref[...] - x_ref[...]
    o_ref[...] = jnp.sum(d * d, axis=2)


_som_distances_tc = pl.pallas_call(
    _som_tc_body,
    grid=(TC_G0 // TCB0,),
    in_specs=[
        pl.BlockSpec((1, 1, D), lambda i: (0, 0, 0)),
        pl.BlockSpec((TCB0, G1, D), lambda i: (SC_G0 // TCB0 + i, 0, 0)),
    ],
    out_specs=pl.BlockSpec((TCB0, G1), lambda i: (i, 0)),
    out_shape=jax.ShapeDtypeStruct((TC_G0, G1), jnp.float32),
)


def kernel(x, weights):
    sc_out = _som_distances_sc(x, weights)
    tc_out = _som_distances_tc(x.reshape(1, 1, D), weights)
    return jnp.concatenate([sc_out, tc_out], axis=0)
